# Initial kernel scaffold; baseline (speedup 1.0000x reference)
import jax
import jax.numpy as jnp
from jax.experimental import pallas as pl


def kernel(x_user, x_travel, x_visit, ei_user_travel, ei_travel_user, ei_travel_visit, ei_visit_travel, params):
    raise NotImplementedError("write your pallas kernel here")


# speedup vs baseline: 1.0000x; 1.0000x over previous
import jax
import jax.numpy as jnp
from jax.experimental import pallas as pl

HID = 128
N_HEADS = 4
NUM_LAYERS = 8


def _linear(x, W, b=None):
    y = x @ W.T
    if b is not None:
        y = y + b
    return y


def _layer_norm(x, g, b, eps=1e-5):
    m = jnp.mean(x, axis=-1, keepdims=True)
    v = jnp.var(x, axis=-1, keepdims=True)
    return (x - m) / jnp.sqrt(v + eps) * g + b


def _sage(x_src, x_dst, ei, p):
    src = ei[0]
    dst = ei[1]
    n_dst = x_dst.shape[0]
    msgs = jnp.take(x_src, src, axis=0)
    s = jax.ops.segment_sum(msgs, dst, num_segments=n_dst)
    cnt = jax.ops.segment_sum(jnp.ones((src.shape[0],), jnp.float32), dst, num_segments=n_dst)
    mean = s / jnp.clip(cnt, 1.0)[:, None]
    return _linear(mean, p['Wl'], p['bl']) + _linear(x_dst, p['Wr'])


def _mha(q, k, v, p):
    N = q.shape[0]
    hd = HID // N_HEADS
    scale = 1.0 / jnp.sqrt(jnp.float32(hd))
    qp = _linear(q, p['Wq'], p['bq']).reshape(N, -1, N_HEADS, hd).transpose(0, 2, 1, 3)
    kp = _linear(k, p['Wk'], p['bk']).reshape(N, -1, N_HEADS, hd).transpose(0, 2, 1, 3)
    vp = _linear(v, p['Wv'], p['bv']).reshape(N, -1, N_HEADS, hd).transpose(0, 2, 1, 3)
    scores = jnp.einsum('nhqd,nhkd->nhqk', qp, kp) * scale
    attn = jax.nn.softmax(scores, axis=-1)
    out = jnp.einsum('nhqk,nhkd->nhqd', attn, vp)
    out = out.transpose(0, 2, 1, 3).reshape(N, -1, HID)
    return _linear(out, p['Wo'], p['bo'])


def kernel(x_user, x_travel, x_visit, ei_user_travel, ei_travel_user, ei_travel_visit, ei_visit_travel, params):
    x = {
        'user': _linear(x_user, params['proj_user']['W'], params['proj_user']['b']),
        'travel': _linear(x_travel, params['proj_travel']['W'], params['proj_travel']['b']),
        'visit_area': x_visit,
    }
    ets = [('user', 'travel', ei_user_travel, 'ut'), ('travel', 'user', ei_travel_user, 'tu'),
           ('travel', 'visit_area', ei_travel_visit, 'tv'), ('visit_area', 'travel', ei_visit_travel, 'vt')]
    for i in range(NUM_LAYERS):
        cp = params['convs'][i]
        h = {nt: jnp.zeros((x[nt].shape[0], HID), jnp.float32) for nt in x}
        for snt, dnt, ei, kk in ets:
            h[dnt] = h[dnt] + _sage(x[snt], x[dnt], ei, cp[kk])
        np_ = params['norms'][i]
        x = {nt: jax.nn.relu(_layer_norm(h[nt], np_[nt]['g'], np_[nt]['b'])) + x[nt] for nt in x}
    hv = x['visit_area']
    q = jnp.broadcast_to(params['attn_query'], (hv.shape[0], HID))[:, None, :]
    attn_out = _mha(q, hv[:, None, :], hv[:, None, :], params['attn'])
    fp = params['final_proj']
    hh = _layer_norm(attn_out[:, 0, :], fp['ln_g'], fp['ln_b'])
    hh = jax.nn.relu(_linear(hh, fp['W1'], fp['b1']))
    score = _linear(hh, fp['W2'], fp['b2'])[:, 0]
    return score


# speedup vs baseline: 1.0228x; 1.0228x over previous
import jax
import jax.numpy as jnp
from jax.experimental import pallas as pl

HID = 128
NUM_LAYERS = 8


def _linear(x, W, b=None):
    y = x @ W.T
    if b is not None:
        y = y + b
    return y


def _layer_norm(x, g, b, eps=1e-5):
    m = jnp.mean(x, axis=-1, keepdims=True)
    v = jnp.var(x, axis=-1, keepdims=True)
    return (x - m) / jnp.sqrt(v + eps) * g + b


def _prep(ei, n_dst):
    src, dst = ei[0], ei[1]
    order = jnp.argsort(dst)
    src_s = src[order]
    dst_s = dst[order]
    cnt = jax.ops.segment_sum(jnp.ones((src.shape[0],), jnp.float32), dst_s,
                              num_segments=n_dst, indices_are_sorted=True)
    inv = 1.0 / jnp.clip(cnt, 1.0)
    return src_s, dst_s, inv


def kernel(x_user, x_travel, x_visit, ei_user_travel, ei_travel_user, ei_travel_visit, ei_visit_travel, params):
    N_U = x_user.shape[0]
    N_T = x_travel.shape[0]
    N_V = x_visit.shape[0]
    prep = {
        'ut': _prep(ei_user_travel, N_T),
        'tu': _prep(ei_travel_user, N_U),
        'tv': _prep(ei_travel_visit, N_V),
        'vt': _prep(ei_visit_travel, N_T),
    }
    x = {
        'user': _linear(x_user, params['proj_user']['W'], params['proj_user']['b']),
        'travel': _linear(x_travel, params['proj_travel']['W'], params['proj_travel']['b']),
        'visit_area': x_visit,
    }
    ets = [('user', 'travel', 'ut'), ('travel', 'user', 'tu'),
           ('travel', 'visit_area', 'tv'), ('visit_area', 'travel', 'vt')]
    for i in range(NUM_LAYERS):
        cp = params['convs'][i]
        h = {nt: None for nt in x}
        for snt, dnt, kk in ets:
            src_s, dst_s, inv = prep[kk]
            p = cp[kk]
            msgs = jnp.take(x[snt], src_s, axis=0)
            s = jax.ops.segment_sum(msgs, dst_s, num_segments=x[dnt].shape[0],
                                    indices_are_sorted=True)
            mean = s * inv[:, None]
            contrib = _linear(mean, p['Wl'], p['bl']) + _linear(x[dnt], p['Wr'])
            h[dnt] = contrib if h[dnt] is None else h[dnt] + contrib
        np_ = params['norms'][i]
        x = {nt: jax.nn.relu(_layer_norm(h[nt], np_[nt]['g'], np_[nt]['b'])) + x[nt] for nt in x}
    hv = x['visit_area']
    ap = params['attn']
    Wc = ap['Wo'] @ ap['Wv']
    bc = ap['bv'] @ ap['Wo'].T + ap['bo']
    attn_out = hv @ Wc.T + bc
    fp = params['final_proj']
    hh = _layer_norm(attn_out, fp['ln_g'], fp['ln_b'])
    hh = jax.nn.relu(_linear(hh, fp['W1'], fp['b1']))
    score = _linear(hh, fp['W2'], fp['b2'])[:, 0]
    return score


# speedup vs baseline: 1.1489x; 1.1233x over previous
import functools

import jax
import jax.numpy as jnp
from jax import lax
from jax.experimental import pallas as pl
from jax.experimental.pallas import tpu as pltpu
from jax.experimental.pallas import tpu_sc as plsc

HID = 128
NUM_LAYERS = 8
G = 128
RB = 512
EPS = 1e-5


@functools.cache
def _sc_segsum(R, NB, NP, n_table, e_pad):
    nzc = R // (16 * 16)
    noc = R // (64 * 16)
    assert R % (64 * 16) == 0 and NB == 2 * NP
    mesh = plsc.VectorSubcoreMesh(core_axis_name="c", subcore_axis_name="s")

    @functools.partial(
        pl.kernel,
        out_type=jax.ShapeDtypeStruct((NB * R, HID), jnp.float32),
        mesh=mesh,
        scratch_types=[
            pltpu.VMEM_SHARED((R + 16, HID), jnp.float32),
            pltpu.VMEM((G,), jnp.int32),
            pltpu.VMEM((G,), jnp.int32),
            pltpu.VMEM((G, HID), jnp.float32),
            pltpu.VMEM((16, HID), jnp.float32),
            pltpu.VMEM((NP * 2 * 16 * 16,), jnp.int32),
            pltpu.SemaphoreType.DMA,
        ],
    )
    def k(table, srcs, dstl, warr, out, acc, src_v, dst_v, rows_v, zbuf, woff,
          sem):
        c = lax.axis_index("c")
        s = lax.axis_index("s")
        pltpu.sync_copy(warr, woff)
        z16 = jnp.zeros((16,), jnp.float32)
        for i in range(16):
            for j in range(HID // 16):
                zbuf[i, pl.ds(j * 16, 16)] = z16
        for p in range(NP):
            b = p * 2 + c
            for kk in range(nzc):
                row0 = (s * nzc + kk) * 16
                pltpu.sync_copy(zbuf, acc.at[pl.ds(row0, 16)])
            plsc.subcore_barrier()
            wv = woff[pl.ds(((p * 2 + c) * 16 + s) * 16, 16)]
            start = wv[0]
            end = wv[1]
            e0 = start - lax.rem(start, 8)
            nch = lax.shift_right_arithmetic(end - e0 + (G - 1), 7)

            def chunk_body(j, carry):
                cb = pl.multiple_of(e0 + j * G, 8)
                pltpu.sync_copy(srcs.at[pl.ds(cb, G)], src_v)
                pltpu.sync_copy(dstl.at[pl.ds(cb, G)], dst_v)
                for t in range(G // 16):
                    pos = cb + t * 16 + lax.iota(jnp.int32, 16)
                    dv = dst_v[pl.ds(t * 16, 16)]
                    ok = (pos >= start) & (pos < end)
                    dst_v[pl.ds(t * 16, 16)] = jnp.where(ok, dv, R)
                pltpu.async_copy(table.at[src_v], rows_v, sem).wait()
                pltpu.sync_copy(rows_v, acc.at[dst_v], add=True)
                return carry

            lax.fori_loop(0, nch, chunk_body, 0)
            plsc.subcore_barrier()
            for kk in range(noc):
                row0 = (s * noc + kk) * 64
                pltpu.sync_copy(acc.at[pl.ds(row0, 64)],
                                out.at[pl.ds(b * R + row0, 64)])

    return k


def _prep_edges(ei, R, NB):
    src, dst = ei[0], ei[1]
    E = src.shape[0]
    D_pad = NB * R
    perm = jnp.argsort(dst)
    src_s = jnp.take(src, perm)
    dst_s = jnp.take(dst, perm)
    ss = jnp.searchsorted(dst_s, jnp.arange(D_pad + 1, dtype=jnp.int32),
                          method='scan').astype(jnp.int32)
    cnt = (ss[1:] - ss[:-1]).astype(jnp.float32)
    inv = (1.0 / jnp.maximum(cnt, 1.0)).reshape(D_pad, 1)
    blk = dst_s // R
    dstl = (dst_s - blk * R).astype(jnp.int32)
    off = ss[jnp.arange(NB + 1) * R]
    cntb = off[1:] - off[:-1]
    sidx = jnp.arange(17, dtype=jnp.int32)
    wstart = off[:NB, None] + (sidx[None, :16] * cntb[:, None]) // 16
    wend = off[:NB, None] + (sidx[None, 1:17] * cntb[:, None]) // 16
    warr = jnp.zeros((NB * 16, 16), jnp.int32)
    warr = warr.at[:, 0].set(wstart.reshape(-1).astype(jnp.int32))
    warr = warr.at[:, 1].set(wend.reshape(-1).astype(jnp.int32))
    pad = 256
    src_p = jnp.concatenate([src_s, jnp.zeros((pad,), jnp.int32)])
    dstl_p = jnp.concatenate([dstl, jnp.full((pad,), R, jnp.int32)])
    return src_p, dstl_p, warr.reshape(-1), inv


def _dotT(x, W):
    return lax.dot_general(x, W, (((1,), (1,)), ((), ())),
                           preferred_element_type=jnp.float32)


@functools.cache
def _tc_linear(n, din, dout):
    def body(xr, wr, br, yr):
        yr[...] = _dotT(xr[...], wr[...]) + br[...]

    return pl.pallas_call(
        body,
        grid=(n // RB,),
        in_specs=[
            pl.BlockSpec((RB, din), lambda i: (i, 0)),
            pl.BlockSpec((dout, din), lambda i: (0, 0)),
            pl.BlockSpec((1, dout), lambda i: (0, 0)),
        ],
        out_specs=pl.BlockSpec((RB, dout), lambda i: (i, 0)),
        out_shape=jax.ShapeDtypeStruct((n, dout), jnp.float32),
    )


def _ln_relu_res(h, xr, gr, br):
    m = jnp.mean(h, axis=-1, keepdims=True)
    v = jnp.mean((h - m) ** 2, axis=-1, keepdims=True)
    y = (h - m) * lax.rsqrt(v + EPS) * gr + br
    return jnp.maximum(y, 0.0) + xr


@functools.cache
def _tc_layer(n, agg_shapes, cuts):
    n_agg = len(agg_shapes)

    def body(*refs):
        xr = refs[0]
        wr = refs[1]
        blc = refs[2]
        gr = refs[3]
        br = refs[4]
        rest = refs[5:5 + 3 * n_agg]
        yr = refs[5 + 3 * n_agg]
        i = pl.program_id(0)
        h = _dotT(xr[...], wr[...]) + blc[...]
        for a in range(n_agg):
            aggr, invr, wlr = rest[3 * a:3 * a + 3]
            mask = jnp.where(i < cuts[a], 1.0, 0.0)
            h = h + mask * (_dotT(aggr[...] * invr[...], wlr[...]))
        yr[...] = _ln_relu_res(h, xr[...], gr[...], br[...])

    in_specs = [
        pl.BlockSpec((RB, HID), lambda i: (i, 0)),
        pl.BlockSpec((HID, HID), lambda i: (0, 0)),
        pl.BlockSpec((1, HID), lambda i: (0, 0)),
        pl.BlockSpec((1, HID), lambda i: (0, 0)),
        pl.BlockSpec((1, HID), lambda i: (0, 0)),
    ]
    for a in range(n_agg):
        cut = cuts[a]
        in_specs += [
            pl.BlockSpec((RB, HID), lambda i, c=cut: (jnp.minimum(i, c - 1), 0)),
            pl.BlockSpec((RB, 1), lambda i, c=cut: (jnp.minimum(i, c - 1), 0)),
            pl.BlockSpec((HID, HID), lambda i: (0, 0)),
        ]
    return pl.pallas_call(
        body,
        grid=(n // RB,),
        in_specs=in_specs,
        out_specs=pl.BlockSpec((RB, HID), lambda i: (i, 0)),
        out_shape=jax.ShapeDtypeStruct((n, HID), jnp.float32),
    )


@functools.cache
def _tc_head(n):
    def body(xr, wcr, bcr, gr, br, w1r, b1r, w2r, b2r, yr):
        z = _dotT(xr[...], wcr[...]) + bcr[...]
        m = jnp.mean(z, axis=-1, keepdims=True)
        v = jnp.mean((z - m) ** 2, axis=-1, keepdims=True)
        zn = (z - m) * lax.rsqrt(v + EPS) * gr[...] + br[...]
        h1 = jnp.maximum(_dotT(zn, w1r[...]) + b1r[...], 0.0)
        yr[...] = jnp.sum(h1 * w2r[...], axis=-1, keepdims=True) + b2r[...]

    return pl.pallas_call(
        body,
        grid=(n // RB,),
        in_specs=[
            pl.BlockSpec((RB, HID), lambda i: (i, 0)),
            pl.BlockSpec((HID, HID), lambda i: (0, 0)),
            pl.BlockSpec((1, HID), lambda i: (0, 0)),
            pl.BlockSpec((1, HID), lambda i: (0, 0)),
            pl.BlockSpec((1, HID), lambda i: (0, 0)),
            pl.BlockSpec((HID, HID), lambda i: (0, 0)),
            pl.BlockSpec((1, HID), lambda i: (0, 0)),
            pl.BlockSpec((1, HID), lambda i: (0, 0)),
            pl.BlockSpec((1, 1), lambda i: (0, 0)),
        ],
        out_specs=pl.BlockSpec((RB, 1), lambda i: (i, 0)),
        out_shape=jax.ShapeDtypeStruct((n, 1), jnp.float32),
    )


def kernel(x_user, x_travel, x_visit, ei_user_travel, ei_travel_user,
           ei_travel_visit, ei_visit_travel, params):
    U, T, V = x_user.shape[0], x_travel.shape[0], x_visit.shape[0]
    R_S, NB_S, NP_S = 5120, 2, 1
    R_B, NB_B, NP_B = 13312, 4, 2
    U_pad, T_pad, V_pad = NB_S * R_S, NB_B * R_B, 100352

    p_ut = _prep_edges(ei_user_travel, R_S, NB_S)
    p_tu = _prep_edges(ei_travel_user, R_S, NB_S)
    p_tv = _prep_edges(ei_travel_visit, R_B, NB_B)
    p_vt = _prep_edges(ei_visit_travel, R_B, NB_B)

    e_ut = p_ut[0].shape[0]
    e_tv = p_tv[0].shape[0]

    xu0 = jnp.pad(x_user, ((0, U_pad - U), (0, 0)))
    xt0 = jnp.pad(x_travel, ((0, T_pad - T), (0, 0)))
    xv = jnp.pad(x_visit, ((0, V_pad - V), (0, 0)))
    pu, pt = params['proj_user'], params['proj_travel']
    xu = _tc_linear(U_pad, 64, HID)(xu0, pu['W'], pu['b'].reshape(1, HID))
    xt = _tc_linear(T_pad, 32, HID)(xt0, pt['W'], pt['b'].reshape(1, HID))

    sc_small_u = _sc_segsum(R_S, NB_S, NP_S, U_pad, e_ut)
    sc_small_t = _sc_segsum(R_S, NB_S, NP_S, T_pad, e_ut)
    sc_big_t = _sc_segsum(R_B, NB_B, NP_B, T_pad, e_tv)
    sc_big_v = _sc_segsum(R_B, NB_B, NP_B, V_pad, e_tv)

    lay_u = _tc_layer(U_pad, (U_pad,), (U_pad // RB,))
    lay_t = _tc_layer(T_pad, (U_pad, T_pad), (U_pad // RB, T_pad // RB))
    lay_v = _tc_layer(V_pad, (T_pad,), (T_pad // RB,))

    for i in range(NUM_LAYERS):
        cp = params['convs'][i]
        np_ = params['norms'][i]
        agg_ut = sc_small_u(xu, p_ut[0], p_ut[1], p_ut[2])
        agg_tu = sc_small_t(xt, p_tu[0], p_tu[1], p_tu[2])
        agg_tv = sc_big_t(xt, p_tv[0], p_tv[1], p_tv[2])
        agg_vt = sc_big_v(xv, p_vt[0], p_vt[1], p_vt[2])

        ptu = cp['tu']
        xu = lay_u(xu, ptu['Wr'], ptu['bl'].reshape(1, HID),
                   np_['user']['g'].reshape(1, HID),
                   np_['user']['b'].reshape(1, HID),
                   agg_tu, p_tu[3], ptu['Wl'])
        put, pvt = cp['ut'], cp['vt']
        wrc = put['Wr'] + pvt['Wr']
        blc = (put['bl'] + pvt['bl']).reshape(1, HID)
        xt = lay_t(xt, wrc, blc,
                   np_['travel']['g'].reshape(1, HID),
                   np_['travel']['b'].reshape(1, HID),
                   agg_ut, p_ut[3], put['Wl'],
                   agg_vt, p_vt[3], pvt['Wl'])
        ptv = cp['tv']
        xv = lay_v(xv, ptv['Wr'], ptv['bl'].reshape(1, HID),
                   np_['visit_area']['g'].reshape(1, HID),
                   np_['visit_area']['b'].reshape(1, HID),
                   agg_tv, p_tv[3], ptv['Wl'])

    ap = params['attn']
    Wc = ap['Wo'] @ ap['Wv']
    bc = (ap['bv'] @ ap['Wo'].T + ap['bo']).reshape(1, HID)
    fp = params['final_proj']
    out = _tc_head(V_pad)(
        xv, Wc, bc,
        fp['ln_g'].reshape(1, HID), fp['ln_b'].reshape(1, HID),
        fp['W1'], fp['b1'].reshape(1, HID),
        fp['W2'], fp['b2'].reshape(1, 1))
    return out[:V, 0]
